# Initial kernel scaffold; baseline (speedup 1.0000x reference)
import functools

import jax
import jax.numpy as jnp
from jax.experimental import pallas as pl
from jax.experimental.pallas import tpu as pltpu

HLEN, WLEN = 128, 128


def _crop_body(hw_ref, bg_ref, out_ref):
    h = hw_ref[0]
    w = hw_ref[1]
    out_ref[...] = bg_ref[:, :, pl.ds(h, HLEN), pl.ds(w, WLEN)]


def _gather_body(ids_ref, table_ref, out_ref, *, bb):
    b = pl.program_id(1)
    for i in range(bb):
        idx = ids_ref[b * bb + i]
        out_ref[i, 0] = table_ref[idx, 0]


def kernel(background, image_id_indices, h, w):
    n_img, c, height, width = background.shape
    batch = image_id_indices.shape[0]

    hw = jnp.stack([jnp.asarray(h, jnp.int32), jnp.asarray(w, jnp.int32)])

    crop = pl.pallas_call(
        _crop_body,
        grid_spec=pltpu.PrefetchScalarGridSpec(
            num_scalar_prefetch=1,
            grid=(n_img,),
            in_specs=[
                pl.BlockSpec((1, c, height, width), lambda i, hw_ref: (i, 0, 0, 0)),
            ],
            out_specs=pl.BlockSpec((1, c, HLEN, WLEN), lambda i, hw_ref: (i, 0, 0, 0)),
        ),
        out_shape=jax.ShapeDtypeStruct((n_img, c, HLEN, WLEN), background.dtype),
    )
    table = crop(hw, background)

    bb = 8
    nb = batch // bb
    gather = pl.pallas_call(
        functools.partial(_gather_body, bb=bb),
        grid_spec=pltpu.PrefetchScalarGridSpec(
            num_scalar_prefetch=1,
            grid=(c, nb),
            in_specs=[
                pl.BlockSpec((n_img, 1, HLEN, WLEN), lambda ci, bi, ids: (0, ci, 0, 0)),
            ],
            out_specs=pl.BlockSpec((bb, 1, HLEN, WLEN), lambda ci, bi, ids: (bi, ci, 0, 0)),
        ),
        out_shape=jax.ShapeDtypeStruct((batch, c, HLEN, WLEN), background.dtype),
    )
    return gather(image_id_indices, table)


# speedup vs baseline: 2.4091x; 2.4091x over previous
import functools

import jax
import jax.numpy as jnp
from jax.experimental import pallas as pl
from jax.experimental.pallas import tpu as pltpu

HLEN, WLEN = 128, 128


def _crop_body(hw_ref, bg_ref, out_ref):
    h = hw_ref[0]
    w = hw_ref[1]
    val = bg_ref[0, 0]
    val = pltpu.roll(val, -h, 0)
    val = pltpu.roll(val, -w, 1)
    out_ref[0, 0] = val[:HLEN, :WLEN]


def _gather_body(ids_ref, table_ref, out_ref, *, bb):
    b = pl.program_id(1)
    for i in range(bb):
        idx = ids_ref[b * bb + i]
        out_ref[i, 0] = table_ref[idx, 0]


def kernel(background, image_id_indices, h, w):
    n_img, c, height, width = background.shape
    batch = image_id_indices.shape[0]

    hw = jnp.stack([jnp.asarray(h, jnp.int32), jnp.asarray(w, jnp.int32)])

    crop = pl.pallas_call(
        _crop_body,
        grid_spec=pltpu.PrefetchScalarGridSpec(
            num_scalar_prefetch=1,
            grid=(n_img, c),
            in_specs=[
                pl.BlockSpec((1, 1, height, width), lambda i, j, hw_ref: (i, j, 0, 0)),
            ],
            out_specs=pl.BlockSpec((1, 1, HLEN, WLEN), lambda i, j, hw_ref: (i, j, 0, 0)),
        ),
        out_shape=jax.ShapeDtypeStruct((n_img, c, HLEN, WLEN), background.dtype),
    )
    table = crop(hw, background)

    bb = 8
    nb = batch // bb
    gather = pl.pallas_call(
        functools.partial(_gather_body, bb=bb),
        grid_spec=pltpu.PrefetchScalarGridSpec(
            num_scalar_prefetch=1,
            grid=(c, nb),
            in_specs=[
                pl.BlockSpec((n_img, 1, HLEN, WLEN), lambda ci, bi, ids: (0, ci, 0, 0)),
            ],
            out_specs=pl.BlockSpec((bb, 1, HLEN, WLEN), lambda ci, bi, ids: (bi, ci, 0, 0)),
        ),
        out_shape=jax.ShapeDtypeStruct((batch, c, HLEN, WLEN), background.dtype),
    )
    return gather(image_id_indices, table)
